# Initial kernel scaffold; baseline (speedup 1.0000x reference)
import jax
import jax.numpy as jnp
from jax.experimental import pallas as pl


def kernel(input, rois):
    raise NotImplementedError("write your pallas kernel here")


# speedup vs baseline: 1.3353x; 1.3353x over previous
import dataclasses
import functools
import math

import jax
import jax.numpy as jnp
from jax import lax
from jax.experimental import pallas as pl
from jax.experimental.pallas import tpu as pltpu
from jax.experimental.pallas import tpu_sc as plsc

POOLED = 7
NBINS = POOLED * POOLED
SCALE = 0.125
NPAD = 1024
T = 32


def _prep_body(n_real, H, W, rois_ref, idx_ref, w_ref):
    r = rois_ref[...]
    bidx = r[0:1, :].astype(jnp.int32)
    cx, cy = r[1:2, :], r[2:3, :]
    hh, ww = r[3:4, :], r[4:5, :]
    th = r[5:6, :] * (math.pi / 180.0)

    Sx = ww * (SCALE / POOLED)
    Sy = hh * (SCALE / POOLED)
    Al, Be = jnp.cos(th), jnp.sin(th)
    dx = dy = -POOLED / 2.0
    M00 = Al * Sx
    M01 = Be * Sy
    M02 = Al * Sx * dx + Be * Sy * dy + cx * SCALE
    M10 = -Be * Sx
    M11 = Al * Sy
    M12 = -Be * Sx * dx + Al * Sy * dy + cy * SCALE

    bi = lax.broadcasted_iota(jnp.int32, (NBINS, NPAD), 0)
    lane = lax.broadcasted_iota(jnp.int32, (NBINS, NPAD), 1)
    pwf = (bi % POOLED).astype(jnp.float32) + 0.5
    phf = (bi // POOLED).astype(jnp.float32) + 0.5
    Px = M00 * pwf + M01 * phf + M02
    Py = M10 * pwf + M11 * phf + M12

    vf = ((Px >= 0.0) & (Px <= W - 1.0) & (Py >= 0.0) & (Py <= H - 1.0)
          & (lane < n_real)).astype(jnp.float32)
    x0i = Px.astype(jnp.int32)
    y0i = Py.astype(jnp.int32)
    wx = Px - x0i.astype(jnp.float32)
    wy = Py - y0i.astype(jnp.float32)
    x0 = jnp.clip(x0i, 0, W - 1)
    x1 = jnp.clip(x0i + 1, 0, W - 1)
    y0 = jnp.clip(y0i, 0, H - 1)
    y1 = jnp.clip(y0i + 1, 0, H - 1)

    base = bidx * (H * W)
    idx_ref[0] = base + y0 * W + x0
    idx_ref[1] = base + y0 * W + x1
    idx_ref[2] = base + y1 * W + x0
    idx_ref[3] = base + y1 * W + x1
    w_ref[0] = (1.0 - wy) * (1.0 - wx) * vf
    w_ref[1] = (1.0 - wy) * wx * vf
    w_ref[2] = wy * (1.0 - wx) * vf
    w_ref[3] = wy * wx * vf


def _sc_pooled_rows(table, idx_g, w_g, C):
    G = idx_g.shape[0]
    K = G * T
    mesh = plsc.VectorSubcoreMesh(core_axis_name="core", subcore_axis_name="subcore")

    cp = pltpu.CompilerParams()
    if "needs_layout_passes" in pltpu.CompilerParams.__dataclass_fields__:
        cp = dataclasses.replace(cp, needs_layout_passes=False)

    @functools.partial(
        pl.kernel,
        out_type=jax.ShapeDtypeStruct((K, C), jnp.float32),
        mesh=mesh,
        scratch_types=[pltpu.VMEM((T, C), jnp.float32) for _ in range(4)],
        compiler_params=cp,
    )
    def sc_kernel(table_hbm, idx_hbm, w_hbm, out_hbm, r0, r1, r2, r3):
        rows = (r0, r1, r2, r3)

        def body(i_vmem, w_vmem, o_vmem):
            for c in range(4):
                pltpu.sync_copy(table_hbm.at[i_vmem.at[0, pl.ds(c * T, T)]],
                                rows[c])

            @pl.loop(0, T)
            def _bin(b):
                bvec = jnp.full((16,), b, jnp.int32)
                zero = jnp.zeros((16,), jnp.int32)
                w0 = plsc.load_gather(w_vmem, [zero, bvec])
                w1 = plsc.load_gather(w_vmem, [zero, bvec + T])
                w2 = plsc.load_gather(w_vmem, [zero, bvec + 2 * T])
                w3 = plsc.load_gather(w_vmem, [zero, bvec + 3 * T])
                for j in range(0, C, 16):
                    s = pl.ds(j, 16)
                    o_vmem[b, s] = (w0 * r0[b, s] + w1 * r1[b, s]
                                    + w2 * r2[b, s] + w3 * r3[b, s])

        pltpu.emit_pipeline(
            body,
            grid=(G,),
            in_specs=[
                pl.BlockSpec((1, 4 * T), lambda i: (i, 0)),
                pl.BlockSpec((1, 4 * T), lambda i: (i, 0)),
            ],
            out_specs=[pl.BlockSpec((T, C), lambda i: (i, 0))],
            core_axis_name=("core", "subcore"),
            dimension_semantics=(pltpu.PARALLEL,),
        )(idx_hbm, w_hbm, out_hbm)

    return sc_kernel(table, idx_g, w_g)


def kernel(input, rois):
    B, C, H, W = input.shape
    n = rois.shape[0]
    assert n <= NPAD

    table = input.transpose(0, 2, 3, 1).reshape(B * H * W, C)
    rois_t = jnp.pad(rois.T, ((0, 0), (0, NPAD - n)))

    idx4, w4 = pl.pallas_call(
        functools.partial(_prep_body, n, H, W),
        out_shape=(
            jax.ShapeDtypeStruct((4, NBINS, NPAD), jnp.int32),
            jax.ShapeDtypeStruct((4, NBINS, NPAD), jnp.float32),
        ),
    )(rois_t)

    K = NBINS * NPAD
    G = K // T
    idx_g = idx4.reshape(4, G, T).transpose(1, 0, 2).reshape(G, 4 * T)
    w_g = w4.reshape(4, G, T).transpose(1, 0, 2).reshape(G, 4 * T)
    out_rows = _sc_pooled_rows(table, idx_g, w_g, C)
    out = out_rows.reshape(NBINS, NPAD, C)[:, :n]
    return out.transpose(1, 2, 0).reshape(n, C, POOLED, POOLED)


# speedup vs baseline: 1.6480x; 1.2342x over previous
import dataclasses
import functools
import math

import jax
import jax.numpy as jnp
from jax import lax
from jax.experimental import pallas as pl
from jax.experimental.pallas import tpu as pltpu
from jax.experimental.pallas import tpu_sc as plsc

POOLED = 7
NBINS = POOLED * POOLED
SCALE = 0.125
NPAD = 1024
T = 32


def _prep_body(n_real, H, W, rois_ref, idx_ref, w_ref):
    r = rois_ref[...]
    bidx = r[0:1, :].astype(jnp.int32)
    cx, cy = r[1:2, :], r[2:3, :]
    hh, ww = r[3:4, :], r[4:5, :]
    th = r[5:6, :] * (math.pi / 180.0)

    Sx = ww * (SCALE / POOLED)
    Sy = hh * (SCALE / POOLED)
    Al, Be = jnp.cos(th), jnp.sin(th)
    dx = dy = -POOLED / 2.0
    M00 = Al * Sx
    M01 = Be * Sy
    M02 = Al * Sx * dx + Be * Sy * dy + cx * SCALE
    M10 = -Be * Sx
    M11 = Al * Sy
    M12 = -Be * Sx * dx + Al * Sy * dy + cy * SCALE

    bi = lax.broadcasted_iota(jnp.int32, (NBINS, NPAD), 0)
    lane = lax.broadcasted_iota(jnp.int32, (NBINS, NPAD), 1)
    pwf = (bi % POOLED).astype(jnp.float32) + 0.5
    phf = (bi // POOLED).astype(jnp.float32) + 0.5
    Px = M00 * pwf + M01 * phf + M02
    Py = M10 * pwf + M11 * phf + M12

    vf = ((Px >= 0.0) & (Px <= W - 1.0) & (Py >= 0.0) & (Py <= H - 1.0)
          & (lane < n_real)).astype(jnp.float32)
    x0i = Px.astype(jnp.int32)
    y0i = Py.astype(jnp.int32)
    wx = Px - x0i.astype(jnp.float32)
    wy = Py - y0i.astype(jnp.float32)
    x0 = jnp.clip(x0i, 0, W - 1)
    x1 = jnp.clip(x0i + 1, 0, W - 1)
    y0 = jnp.clip(y0i, 0, H - 1)
    y1 = jnp.clip(y0i + 1, 0, H - 1)

    base = bidx * (H * W)
    idx_ref[0] = base + y0 * W + x0
    idx_ref[1] = base + y0 * W + x1
    idx_ref[2] = base + y1 * W + x0
    idx_ref[3] = base + y1 * W + x1
    w_ref[0] = (1.0 - wy) * (1.0 - wx) * vf
    w_ref[1] = (1.0 - wy) * wx * vf
    w_ref[2] = wy * (1.0 - wx) * vf
    w_ref[3] = wy * wx * vf


def _sc_pooled_rows(table, idx_g, w_g, C):
    G = idx_g.shape[0]
    K = G * T
    mesh = plsc.VectorSubcoreMesh(core_axis_name="core", subcore_axis_name="subcore")

    cp = pltpu.CompilerParams()
    if "needs_layout_passes" in pltpu.CompilerParams.__dataclass_fields__:
        cp = dataclasses.replace(cp, needs_layout_passes=False)

    @functools.partial(
        pl.kernel,
        out_type=jax.ShapeDtypeStruct((K, C), jnp.float32),
        mesh=mesh,
        scratch_types=[pltpu.VMEM((T, C), jnp.float32) for _ in range(4)]
        + [pltpu.SemaphoreType.DMA],
        compiler_params=cp,
    )
    def sc_kernel(table_hbm, idx_hbm, w_hbm, out_hbm, r0, r1, r2, r3, sem):
        rows = (r0, r1, r2, r3)

        def body(i_vmem, w_vmem, o_vmem):
            copies = [
                pltpu.async_copy(table_hbm.at[i_vmem.at[0, pl.ds(c * T, T)]],
                                 rows[c], sem)
                for c in range(4)
            ]
            for copy in copies:
                copy.wait()

            @pl.loop(0, T)
            def _bin(b):
                bvec = jnp.full((16,), b, jnp.int32)
                zero = jnp.zeros((16,), jnp.int32)
                w0 = plsc.load_gather(w_vmem, [zero, bvec])
                w1 = plsc.load_gather(w_vmem, [zero, bvec + T])
                w2 = plsc.load_gather(w_vmem, [zero, bvec + 2 * T])
                w3 = plsc.load_gather(w_vmem, [zero, bvec + 3 * T])
                for j in range(0, C, 16):
                    s = pl.ds(j, 16)
                    o_vmem[b, s] = (w0 * r0[b, s] + w1 * r1[b, s]
                                    + w2 * r2[b, s] + w3 * r3[b, s])

        pltpu.emit_pipeline(
            body,
            grid=(G,),
            in_specs=[
                pl.BlockSpec((1, 4 * T), lambda i: (i, 0)),
                pl.BlockSpec((1, 4 * T), lambda i: (i, 0)),
            ],
            out_specs=[pl.BlockSpec((T, C), lambda i: (i, 0))],
            core_axis_name=("core", "subcore"),
            dimension_semantics=(pltpu.PARALLEL,),
        )(idx_hbm, w_hbm, out_hbm)

    return sc_kernel(table, idx_g, w_g)


def kernel(input, rois):
    B, C, H, W = input.shape
    n = rois.shape[0]
    assert n <= NPAD

    table = input.transpose(0, 2, 3, 1).reshape(B * H * W, C)
    rois_t = jnp.pad(rois.T, ((0, 0), (0, NPAD - n)))

    idx4, w4 = pl.pallas_call(
        functools.partial(_prep_body, n, H, W),
        out_shape=(
            jax.ShapeDtypeStruct((4, NBINS, NPAD), jnp.int32),
            jax.ShapeDtypeStruct((4, NBINS, NPAD), jnp.float32),
        ),
    )(rois_t)

    K = NBINS * NPAD
    G = K // T
    idx_g = idx4.reshape(4, G, T).transpose(1, 0, 2).reshape(G, 4 * T)
    w_g = w4.reshape(4, G, T).transpose(1, 0, 2).reshape(G, 4 * T)
    out_rows = _sc_pooled_rows(table, idx_g, w_g, C)
    out = out_rows.reshape(NBINS, NPAD, C)[:, :n]
    return out.transpose(1, 2, 0).reshape(n, C, POOLED, POOLED)
